# baseline (speedup 1.0000x reference)
import jax
import jax.numpy as jnp
from jax.experimental import pallas as pl
from jax.experimental.pallas import tpu as pltpu

_GATHER_BETA = 3.0
_SCATTER_BETA = 1e10


def _body(lg_ref, sh_ref, sc_ref, nh_ref, ncv_ref,
          gh_ref, gc_ref, uh_ref, uc_ref):
    nc = sh_ref.shape[0]
    x = lg_ref[...]
    xg = x * _GATHER_BETA
    xg = xg - jnp.max(xg, axis=-1, keepdims=True)
    eg = jnp.exp(xg)
    wg = eg / jnp.sum(eg, axis=-1, keepdims=True)
    xs = x * _SCATTER_BETA
    xs = xs - jnp.max(xs, axis=-1, keepdims=True)
    es = jnp.exp(xs)
    ws = es / jnp.sum(es, axis=-1, keepdims=True)

    nh = nh_ref[0]
    ncv = ncv_ref[0]
    acc_h = None
    acc_c = None
    for c in range(nc):
        sh = sh_ref[c, 0]
        sc = sc_ref[c, 0]
        wgc = wg[0, 0, c]
        wsc = ws[0, 0, c]
        uh_ref[c, 0] = sh * (1.0 - wsc) + nh * wsc
        uc_ref[c, 0] = sc * (1.0 - wsc) + ncv * wsc
        ch = sh * wgc
        cc = sc * wgc
        acc_h = ch if acc_h is None else acc_h + ch
        acc_c = cc if acc_c is None else acc_c + cc
    gh_ref[0] = acc_h
    gc_ref[0] = acc_c


def kernel(states_h, states_c, new_h, new_c, logits):
    nc, bs, h, w, oc = states_h.shape
    bh = 8

    lg = logits.reshape(bs, 1, nc)

    grid = (bs, h // bh)
    states_spec = pl.BlockSpec((nc, 1, bh, w, oc), lambda b, i: (0, b, i, 0, 0))
    plane_spec = pl.BlockSpec((1, bh, w, oc), lambda b, i: (b, i, 0, 0))
    gh, gc, uh, uc = pl.pallas_call(
        _body,
        grid=grid,
        in_specs=[
            pl.BlockSpec((1, 1, nc), lambda b, i: (b, 0, 0)),
            states_spec,
            states_spec,
            plane_spec,
            plane_spec,
        ],
        out_specs=[plane_spec, plane_spec, states_spec, states_spec],
        out_shape=(
            jax.ShapeDtypeStruct((bs, h, w, oc), states_h.dtype),
            jax.ShapeDtypeStruct((bs, h, w, oc), states_h.dtype),
            jax.ShapeDtypeStruct((nc, bs, h, w, oc), states_h.dtype),
            jax.ShapeDtypeStruct((nc, bs, h, w, oc), states_h.dtype),
        ),
        compiler_params=pltpu.CompilerParams(
            dimension_semantics=("parallel", "parallel"),
            vmem_limit_bytes=56 * 1024 * 1024,
        ),
    )(lg, states_h, states_c, new_h, new_c)

    return gh, gc, uh, uc


# speedup vs baseline: 6.5252x; 6.5252x over previous
import jax
import jax.numpy as jnp
from jax.experimental import pallas as pl
from jax.experimental.pallas import tpu as pltpu

_GATHER_BETA = 3.0
_SCATTER_BETA = 1e10


def _body(lg_ref, sh_ref, sc_ref, nh_ref, ncv_ref,
          gh_ref, gc_ref, uh_ref, uc_ref):
    nc = sh_ref.shape[0]
    x = lg_ref[...]
    xg = x * _GATHER_BETA
    xg = xg - jnp.max(xg, axis=-1, keepdims=True)
    eg = jnp.exp(xg)
    wg = eg / jnp.sum(eg, axis=-1, keepdims=True)
    xs = x * _SCATTER_BETA
    xs = xs - jnp.max(xs, axis=-1, keepdims=True)
    es = jnp.exp(xs)
    ws = es / jnp.sum(es, axis=-1, keepdims=True)

    nh = nh_ref[0]
    ncv = ncv_ref[0]
    acc_h = None
    acc_c = None
    for c in range(nc):
        sh = sh_ref[c, 0]
        sc = sc_ref[c, 0]
        wgc = wg[0, 0, c]
        wsc = ws[0, 0, c]
        uh_ref[c, 0] = sh * (1.0 - wsc) + nh * wsc
        uc_ref[c, 0] = sc * (1.0 - wsc) + ncv * wsc
        ch = sh * wgc
        cc = sc * wgc
        acc_h = ch if acc_h is None else acc_h + ch
        acc_c = cc if acc_c is None else acc_c + cc
    gh_ref[0] = acc_h
    gc_ref[0] = acc_c


def kernel(states_h, states_c, new_h, new_c, logits):
    nc, bs, h, w, oc = states_h.shape
    bh = 8

    lg = logits.reshape(bs, 1, nc)
    sh_t = jnp.swapaxes(states_h, 3, 4)
    sc_t = jnp.swapaxes(states_c, 3, 4)
    nh_t = jnp.swapaxes(new_h, 2, 3)
    ncv_t = jnp.swapaxes(new_c, 2, 3)

    grid = (bs, h // bh)
    states_spec = pl.BlockSpec((nc, 1, bh, oc, w), lambda b, i: (0, b, i, 0, 0))
    plane_spec = pl.BlockSpec((1, bh, oc, w), lambda b, i: (b, i, 0, 0))
    gh, gc, uh, uc = pl.pallas_call(
        _body,
        grid=grid,
        in_specs=[
            pl.BlockSpec((1, 1, nc), lambda b, i: (b, 0, 0)),
            states_spec,
            states_spec,
            plane_spec,
            plane_spec,
        ],
        out_specs=[plane_spec, plane_spec, states_spec, states_spec],
        out_shape=(
            jax.ShapeDtypeStruct((bs, h, oc, w), states_h.dtype),
            jax.ShapeDtypeStruct((bs, h, oc, w), states_h.dtype),
            jax.ShapeDtypeStruct((nc, bs, h, oc, w), states_h.dtype),
            jax.ShapeDtypeStruct((nc, bs, h, oc, w), states_h.dtype),
        ),
        compiler_params=pltpu.CompilerParams(
            dimension_semantics=("parallel", "parallel"),
            vmem_limit_bytes=56 * 1024 * 1024,
        ),
    )(lg, sh_t, sc_t, nh_t, ncv_t)

    return (jnp.swapaxes(gh, 2, 3), jnp.swapaxes(gc, 2, 3),
            jnp.swapaxes(uh, 3, 4), jnp.swapaxes(uc, 3, 4))


# speedup vs baseline: 7.9866x; 1.2240x over previous
import jax
import jax.numpy as jnp
from jax.experimental import pallas as pl
from jax.experimental.pallas import tpu as pltpu

_GATHER_BETA = 3.0
_SCATTER_BETA = 1e10


def _body(lg_ref, sh_ref, sc_ref, nh_ref, ncv_ref,
          gh_ref, gc_ref, uh_ref, uc_ref):
    nc = sh_ref.shape[0]
    x = lg_ref[...]
    xg = x * _GATHER_BETA
    xg = xg - jnp.max(xg, axis=-1, keepdims=True)
    eg = jnp.exp(xg)
    wg = eg / jnp.sum(eg, axis=-1, keepdims=True)
    xs = x * _SCATTER_BETA
    xs = xs - jnp.max(xs, axis=-1, keepdims=True)
    es = jnp.exp(xs)
    ws = es / jnp.sum(es, axis=-1, keepdims=True)

    nh = nh_ref[0]
    ncv = ncv_ref[0]
    acc_h = None
    acc_c = None
    for c in range(nc):
        sh = sh_ref[c, 0]
        sc = sc_ref[c, 0]
        wgc = wg[0, 0, c]
        wsc = ws[0, 0, c]
        uh_ref[c, 0] = sh * (1.0 - wsc) + nh * wsc
        uc_ref[c, 0] = sc * (1.0 - wsc) + ncv * wsc
        ch = sh * wgc
        cc = sc * wgc
        acc_h = ch if acc_h is None else acc_h + ch
        acc_c = cc if acc_c is None else acc_c + cc
    gh_ref[0] = acc_h
    gc_ref[0] = acc_c


def kernel(states_h, states_c, new_h, new_c, logits):
    nc, bs, h, w, oc = states_h.shape
    bh = 16

    lg = logits.reshape(bs, 1, nc)
    sh_t = jnp.swapaxes(states_h, 3, 4)
    sc_t = jnp.swapaxes(states_c, 3, 4)
    nh_t = jnp.swapaxes(new_h, 2, 3)
    ncv_t = jnp.swapaxes(new_c, 2, 3)

    grid = (bs, h // bh)
    states_spec = pl.BlockSpec((nc, 1, bh, oc, w), lambda b, i: (0, b, i, 0, 0))
    plane_spec = pl.BlockSpec((1, bh, oc, w), lambda b, i: (b, i, 0, 0))
    gh, gc, uh, uc = pl.pallas_call(
        _body,
        grid=grid,
        in_specs=[
            pl.BlockSpec((1, 1, nc), lambda b, i: (b, 0, 0)),
            states_spec,
            states_spec,
            plane_spec,
            plane_spec,
        ],
        out_specs=[plane_spec, plane_spec, states_spec, states_spec],
        out_shape=(
            jax.ShapeDtypeStruct((bs, h, oc, w), states_h.dtype),
            jax.ShapeDtypeStruct((bs, h, oc, w), states_h.dtype),
            jax.ShapeDtypeStruct((nc, bs, h, oc, w), states_h.dtype),
            jax.ShapeDtypeStruct((nc, bs, h, oc, w), states_h.dtype),
        ),
        compiler_params=pltpu.CompilerParams(
            dimension_semantics=("parallel", "parallel"),
            vmem_limit_bytes=56 * 1024 * 1024,
        ),
    )(lg, sh_t, sc_t, nh_t, ncv_t)

    return (jnp.swapaxes(gh, 2, 3), jnp.swapaxes(gc, 2, 3),
            jnp.swapaxes(uh, 3, 4), jnp.swapaxes(uc, 3, 4))
